# baseline (speedup 1.0000x reference)
import functools

import jax
import jax.numpy as jnp
from jax import lax
from jax.experimental import pallas as pl
from jax.experimental.pallas import tpu as pltpu
from jax.experimental.pallas import tpu_sc as plsc

_ROWS = 16384
_COLS = 4096
_N = _ROWS * _COLS
_SPLIT = 12288
_NC = 2
_NS = 16
_NW = _NC * _NS
_SCROWS = _ROWS - _SPLIT
_WROWS = _SCROWS // _NW
_CR = 8
_CC = 2048
_C = _CR * _CC
_NCHUNK = (_WROWS // _CR) * (_COLS // _CC)
_GPC = _C // 64
_TCBLK = 256

assert _SCROWS % (_NW * _CR) == 0
assert _SPLIT % _TCBLK == 0

_mesh = plsc.VectorSubcoreMesh(core_axis_name="c", subcore_axis_name="s")


@functools.partial(
    pl.kernel,
    mesh=_mesh,
    compiler_params=pltpu.CompilerParams(needs_layout_passes=False),
    out_type=(
        jax.ShapeDtypeStruct((_NW, 16), jnp.float32),
        jax.ShapeDtypeStruct((_NW, 16), jnp.int32),
    ),
    scratch_types=[
        pltpu.VMEM((_CR, _CC), jnp.float32),
        pltpu.VMEM((_CR, _CC), jnp.float32),
        pltpu.VMEM((_CR, _CC), jnp.float32),
        pltpu.VMEM((_CR, _CC), jnp.float32),
        pltpu.VMEM((_C,), jnp.int8),
        pltpu.VMEM((_C,), jnp.int8),
        pltpu.VMEM((16,), jnp.float32),
        pltpu.VMEM((16,), jnp.int32),
        pltpu.SemaphoreType.DMA,
        pltpu.SemaphoreType.DMA,
        pltpu.SemaphoreType.DMA,
        pltpu.SemaphoreType.DMA,
        pltpu.SemaphoreType.DMA,
        pltpu.SemaphoreType.DMA,
    ],
)
def _sc_loss(p_hbm, t_hbm, m_hbm, sum_hbm, cnt_hbm,
             p0, p1, t0, t1, m0, m1, osum, ocnt,
             sp0, sp1, st0, st1, sm0, sm1):
    wid = lax.axis_index("s") * _NC + lax.axis_index("c")
    row0 = _SPLIT + wid * _WROWS

    pbuf = (p0, p1)
    tbuf = (t0, t1)
    mbuf = (m0, m1)
    psem = (sp0, sp1)
    tsem = (st0, st1)
    msem = (sm0, sm1)

    def _copies(slot, chunk):
        rb = chunk >> 1
        ch0 = (chunk & 1) * _CC
        r = row0 + rb * _CR
        out = [
            pltpu.make_async_copy(
                p_hbm.at[pl.ds(r, _CR), pl.ds(ch0, _CC)],
                pbuf[slot], psem[slot]),
            pltpu.make_async_copy(
                t_hbm.at[pl.ds(r, _CR), pl.ds(ch0, _CC)],
                tbuf[slot], tsem[slot]),
        ]
        mrow0 = (r - _SPLIT) * _COLS + ch0
        for k in range(_CR):
            out.append(pltpu.make_async_copy(
                m_hbm.at[pl.ds(mrow0 + k * _COLS, _CC)],
                mbuf[slot].at[pl.ds(k * _CC, _CC)], msem[slot]))
        return out

    def _issue(slot, chunk):
        for c in _copies(slot, chunk):
            c.start()

    def _wait(slot, chunk):
        for c in _copies(slot, chunk):
            c.wait()

    iota = lax.iota(jnp.int32, 16)
    offs = tuple(iota * 4 + b for b in range(4))
    ones_w = jnp.int32(0x01010101)
    zeros16 = jnp.zeros((16,), jnp.int32)

    def _consume(slot, acc, cnt):
        pb, tb, mb = pbuf[slot], tbuf[slot], mbuf[slot]

        def group(g, carry):
            a, c = carry
            mv8 = mb[pl.ds(g * 64, 64)]
            mw = plsc.bitcast(mv8, jnp.int32)
            c = c + ((mw & ones_w) * ones_w >> 24)
            rowv = zeros16 + (g >> 5)
            c0 = (g & 31) * 64
            for b in range(4):
                bits = (mw >> (8 * b)) & 1 if b else mw & 1
                keep = bits == 0
                colv = offs[b] + c0
                pv = plsc.load_gather(pb, [rowv, colv])
                tv = plsc.load_gather(tb, [rowv, colv])
                d = pv - tv
                a = a + jnp.where(keep, d * d, jnp.float32(0.0))
            return a, c

        return lax.fori_loop(0, _GPC, group, (acc, cnt))

    acc = jnp.zeros((16,), jnp.float32)
    cnt = jnp.zeros((16,), jnp.int32)

    _issue(0, 0)

    def body(i, carry):
        acc, cnt = carry
        c0 = i * 2
        _issue(1, c0 + 1)
        _wait(0, c0)
        acc, cnt = _consume(0, acc, cnt)

        @pl.when(c0 + 2 < _NCHUNK)
        def _():
            _issue(0, c0 + 2)

        _wait(1, c0 + 1)
        acc, cnt = _consume(1, acc, cnt)
        return acc, cnt

    acc, cnt = lax.fori_loop(0, _NCHUNK // 2, body, (acc, cnt))

    osum[...] = acc
    ocnt[...] = cnt
    pltpu.sync_copy(osum, sum_hbm.at[wid])
    pltpu.sync_copy(ocnt, cnt_hbm.at[wid])


def _tc_body(pred_ref, target_ref, mask_ref, sum_ref, cnt_ref, accs_ref, accc_ref):
    i = pl.program_id(0)

    @pl.when(i == 0)
    def _init():
        accs_ref[0] = jnp.float32(0.0)
        accc_ref[0] = jnp.int32(0)

    p = pred_ref[...]
    t = target_ref[...]
    m = mask_ref[...]
    d = p - t
    d2 = d * d
    d2m = jnp.where(m, jnp.float32(jnp.nan), d2)
    valid = d2m == d2m
    accs_ref[0] += jnp.sum(jnp.where(valid, d2, jnp.float32(0.0)))
    accc_ref[0] += jnp.sum(valid.astype(jnp.int32))

    @pl.when(i == pl.num_programs(0) - 1)
    def _fini():
        sum_ref[0, 0] = accs_ref[0]
        cnt_ref[0, 0] = accc_ref[0]


def _tc_loss(p2, t2, m2):
    grid = (_SPLIT // _TCBLK,)
    return pl.pallas_call(
        _tc_body,
        grid=grid,
        in_specs=[
            pl.BlockSpec((_TCBLK, _COLS), lambda i: (i, 0)),
            pl.BlockSpec((_TCBLK, _COLS), lambda i: (i, 0)),
            pl.BlockSpec((_TCBLK, _COLS), lambda i: (i, 0)),
        ],
        out_specs=(
            pl.BlockSpec(memory_space=pltpu.SMEM),
            pl.BlockSpec(memory_space=pltpu.SMEM),
        ),
        out_shape=(
            jax.ShapeDtypeStruct((1, 1), jnp.float32),
            jax.ShapeDtypeStruct((1, 1), jnp.int32),
        ),
        scratch_shapes=[
            pltpu.SMEM((1,), jnp.float32),
            pltpu.SMEM((1,), jnp.int32),
        ],
        compiler_params=pltpu.CompilerParams(
            dimension_semantics=("arbitrary",),
        ),
    )(p2, t2, m2)


def kernel(pred, target, mask):
    p2 = pred.reshape(_ROWS, _COLS)
    t2 = target.reshape(_ROWS, _COLS)
    m2 = mask.reshape(_ROWS, _COLS)
    m8 = m2[_SPLIT:].view(jnp.int8).reshape(_SCROWS * _COLS)
    sc_sums, sc_cnts = _sc_loss(p2, t2, m8)
    tc_sum, tc_cnt = _tc_loss(p2, t2, m2)
    total = tc_sum[0, 0] + jnp.sum(sc_sums)
    sc_valid = jnp.int32(_SCROWS * _COLS) - jnp.sum(sc_cnts)
    count = (tc_cnt[0, 0] + sc_valid).astype(jnp.float32)
    return total / count


# speedup vs baseline: 1.5709x; 1.5709x over previous
import jax
import jax.numpy as jnp
from jax.experimental import pallas as pl
from jax.experimental.pallas import tpu as pltpu

_ROWS = 16384
_COLS = 4096
_BLOCK_ROWS = 256


def _body(pred_ref, target_ref, mask_ref, out_ref, sum_ref, cnt_ref):
    i = pl.program_id(0)

    @pl.when(i == 0)
    def _init():
        sum_ref[0] = jnp.float32(0.0)
        cnt_ref[0] = jnp.int32(0)

    p = pred_ref[...]
    t = target_ref[...]
    m = mask_ref[...]
    d = p - t
    d2 = d * d
    d2m = jnp.where(m != 0, jnp.float32(jnp.nan), d2)
    valid = d2m == d2m
    sum_ref[0] += jnp.sum(jnp.where(valid, d2, jnp.float32(0.0)))
    cnt_ref[0] += jnp.sum(valid.astype(jnp.int32))

    @pl.when(i == pl.num_programs(0) - 1)
    def _fini():
        out_ref[0, 0] = sum_ref[0] / cnt_ref[0].astype(jnp.float32)


def kernel(pred, target, mask):
    p = pred.reshape(_ROWS, _COLS)
    t = target.reshape(_ROWS, _COLS)
    m = mask.view(jnp.int8).reshape(_ROWS, _COLS)
    grid = (_ROWS // _BLOCK_ROWS,)
    out = pl.pallas_call(
        _body,
        grid=grid,
        in_specs=[
            pl.BlockSpec((_BLOCK_ROWS, _COLS), lambda i: (i, 0)),
            pl.BlockSpec((_BLOCK_ROWS, _COLS), lambda i: (i, 0)),
            pl.BlockSpec((_BLOCK_ROWS, _COLS), lambda i: (i, 0)),
        ],
        out_specs=pl.BlockSpec(memory_space=pltpu.SMEM),
        out_shape=jax.ShapeDtypeStruct((1, 1), jnp.float32),
        scratch_shapes=[
            pltpu.SMEM((1,), jnp.float32),
            pltpu.SMEM((1,), jnp.int32),
        ],
        compiler_params=pltpu.CompilerParams(
            dimension_semantics=("arbitrary",),
        ),
    )(p, t, m)
    return out.reshape(())
